# Initial kernel scaffold; baseline (speedup 1.0000x reference)
import jax
import jax.numpy as jnp
from jax.experimental import pallas as pl


def kernel(layers_num, user_emb, entity_emb, inter_edge, inter_edge_w, edge_index, edge_type, relation_emb, W_Q):
    raise NotImplementedError("write your pallas kernel here")


# speedup vs baseline: 2.4301x; 2.4301x over previous
import dataclasses

import jax
import jax.numpy as jnp
from jax import lax
from jax.experimental import pallas as pl
from jax.experimental.pallas import tpu as pltpu
from jax.experimental.pallas import tpu_sc as plsc

N_USERS = 8192
N_ENTITIES = 10000
DIMS = 128
N_HEADS = 2
D_K = DIMS // N_HEADS
N_EDGES = 320000
N_INTER = 400000
N_RELATIONS = 16
LAYERS = 2

NSUB = 16
B = 64

E_ROWS = 10112
U_ROWS = 8320
DEN_ROWS = 256

EB_PER_SUB = 313
E_PER_SUB = EB_PER_SUB * B
NEP = E_PER_SUB * NSUB
UB_PER_SUB = 391
U_PER_SUB = UB_PER_SUB * B
NUP = U_PER_SUB * NSUB

ER_PER_SUB = E_ROWS // NSUB
UR_PER_SUB = U_ROWS // NSUB
DR_PER_SUB = DEN_ROWS // NSUB


def _sc_agg(p_tab, e_tab, r2, headp, tailp, etp, iu, ii, iw, zrows):
    mesh = plsc.VectorSubcoreMesh(core_axis_name="c", subcore_axis_name="s")

    def body(p_hbm, e_hbm, r2_hbm, head_hbm, tail_hbm, et_hbm,
             iu_hbm, ii_hbm, iw_hbm, z_hbm, acc_out, den_out, uacc_out,
             acc_sh, den_sh, hid_v, tid_v, et_v, didx_v, wf_v,
             q_v, k_v, v_v, r_v, sem):
        cid = lax.axis_index("c")
        sid = lax.axis_index("s")

        z0 = sid * ER_PER_SUB
        pltpu.sync_copy(z_hbm.at[pl.ds(z0, ER_PER_SUB)],
                        acc_sh.at[pl.ds(z0, ER_PER_SUB)])
        d0_ = sid * DR_PER_SUB
        pltpu.sync_copy(z_hbm.at[pl.ds(d0_, DR_PER_SUB)],
                        den_sh.at[pl.ds(d0_, DR_PER_SUB)])
        plsc.subcore_barrier()

        lane = lax.iota(jnp.int32, 16)

        @pl.when(cid == 0)
        def _entity():
            @pl.loop(0, EB_PER_SUB)
            def _(b):
                base = sid * E_PER_SUB + b * B
                pltpu.sync_copy(head_hbm.at[pl.ds(base, B)], hid_v)
                pltpu.sync_copy(tail_hbm.at[pl.ds(base, B)], tid_v)
                pltpu.sync_copy(et_hbm.at[pl.ds(base, B)], et_v)
                d1 = pltpu.async_copy(p_hbm.at[hid_v], q_v, sem)
                d2 = pltpu.async_copy(p_hbm.at[tid_v], k_v, sem)
                d3 = pltpu.async_copy(e_hbm.at[tid_v], v_v, sem)
                d4 = pltpu.async_copy(r2_hbm.at[et_v], r_v, sem)
                for c in range(B // 16):
                    sl = pl.ds(c * 16, 16)
                    didx_v[sl] = lax.shift_right_logical(hid_v[sl], 6)
                d1.wait(); d2.wait(); d3.wait(); d4.wait()

                @pl.loop(0, B)
                def _(e):
                    s0 = jnp.float32(0.0)
                    s1 = jnp.float32(0.0)
                    for c in range(8):
                        sl = pl.ds(c * 16, 16)
                        kc = k_v[e, sl] * r_v[e, sl]
                        t = jnp.sum(q_v[e, sl] * kc)
                        if c < 4:
                            s0 = s0 + t
                        else:
                            s1 = s1 + t
                    ex0 = jnp.exp(jnp.full((16,), s0 * 0.125, jnp.float32))
                    ex1 = jnp.exp(jnp.full((16,), s1 * 0.125, jnp.float32))
                    for c in range(8):
                        sl = pl.ds(c * 16, 16)
                        vc = v_v[e, sl] * r_v[e, sl]
                        k_v[e, sl] = vc * (ex0 if c < 4 else ex1)
                    nsp = plsc.load_gather(
                        hid_v, [jnp.full((16,), e, jnp.int32)])
                    t0 = lax.shift_left(
                        jnp.bitwise_and(nsp, jnp.int32(63)), 1)
                    for c in range(8):
                        sl = pl.ds(c * 16, 16)
                        dl = lane + c * 16
                        q_v[e, sl] = jnp.where(
                            dl == t0, ex0,
                            jnp.where(dl == t0 + 1, ex1, 0.0))

                pltpu.sync_copy(k_v, acc_sh.at[hid_v], add=True)
                pltpu.sync_copy(q_v, den_sh.at[didx_v], add=True)

        @pl.when(cid == 1)
        def _user():
            @pl.loop(0, UB_PER_SUB)
            def _(b):
                base = sid * U_PER_SUB + b * B
                pltpu.sync_copy(iu_hbm.at[pl.ds(base, B)], hid_v)
                pltpu.sync_copy(ii_hbm.at[pl.ds(base, B)], tid_v)
                pltpu.sync_copy(iw_hbm.at[pl.ds(base, B)], wf_v)
                pltpu.async_copy(e_hbm.at[tid_v], v_v, sem).wait()

                @pl.loop(0, B)
                def _(e):
                    wsp = plsc.load_gather(
                        wf_v, [jnp.full((16,), e, jnp.int32)])
                    for c in range(8):
                        sl = pl.ds(c * 16, 16)
                        v_v[e, sl] = v_v[e, sl] * wsp

                pltpu.sync_copy(v_v, acc_sh.at[hid_v], add=True)

        plsc.subcore_barrier()

        @pl.when(cid == 0)
        def _out_e():
            r0 = sid * ER_PER_SUB
            pltpu.sync_copy(acc_sh.at[pl.ds(r0, ER_PER_SUB)],
                            acc_out.at[pl.ds(r0, ER_PER_SUB)])
            r1 = sid * DR_PER_SUB
            pltpu.sync_copy(den_sh.at[pl.ds(r1, DR_PER_SUB)],
                            den_out.at[pl.ds(r1, DR_PER_SUB)])

        @pl.when(cid == 1)
        def _out_u():
            r0 = sid * UR_PER_SUB
            pltpu.sync_copy(acc_sh.at[pl.ds(r0, UR_PER_SUB)],
                            uacc_out.at[pl.ds(r0, UR_PER_SUB)])

    cp = pltpu.CompilerParams()
    if "needs_layout_passes" in pltpu.CompilerParams.__dataclass_fields__:
        cp = dataclasses.replace(cp, needs_layout_passes=False)
    run = pl.kernel(
        body,
        out_type=(jax.ShapeDtypeStruct((E_ROWS, DIMS), jnp.float32),
                  jax.ShapeDtypeStruct((DEN_ROWS, DIMS), jnp.float32),
                  jax.ShapeDtypeStruct((U_ROWS, DIMS), jnp.float32)),
        mesh=mesh,
        compiler_params=cp,
        scratch_types=[
            pltpu.VMEM_SHARED((E_ROWS, DIMS), jnp.float32),
            pltpu.VMEM_SHARED((DEN_ROWS, DIMS), jnp.float32),
            pltpu.VMEM((B,), jnp.int32),
            pltpu.VMEM((B,), jnp.int32),
            pltpu.VMEM((B,), jnp.int32),
            pltpu.VMEM((B,), jnp.int32),
            pltpu.VMEM((B,), jnp.float32),
            pltpu.VMEM((B, DIMS), jnp.float32),
            pltpu.VMEM((B, DIMS), jnp.float32),
            pltpu.VMEM((B, DIMS), jnp.float32),
            pltpu.VMEM((B, DIMS), jnp.float32),
            pltpu.SemaphoreType.DMA,
        ],
    )
    return run(p_tab, e_tab, r2, headp, tailp, etp, iu, ii, iw, zrows)


def _tc_project(e_tab, w_q):
    def mm(e_ref, w_ref, o_ref):
        o_ref[...] = jnp.dot(e_ref[...], w_ref[...],
                             preferred_element_type=jnp.float32)

    return pl.pallas_call(
        mm,
        grid=(4,),
        in_specs=[pl.BlockSpec((E_ROWS // 4, DIMS), lambda i: (i, 0)),
                  pl.BlockSpec((DIMS, DIMS), lambda i: (0, 0))],
        out_specs=pl.BlockSpec((E_ROWS // 4, DIMS), lambda i: (i, 0)),
        out_shape=jax.ShapeDtypeStruct((E_ROWS, DIMS), jnp.float32),
    )(e_tab, w_q)


def _tc_finalize(acc, den2):
    def fin(a_ref, d_ref, o_ref):
        val = a_ref[...]
        d0 = d_ref[:, 0:1]
        d1 = d_ref[:, 1:2]
        lane = lax.broadcasted_iota(jnp.int32, val.shape, 1)
        den = jnp.where(lane < D_K, d0, d1)
        agg = val / (den + 1e-16)
        norm = jnp.sqrt(jnp.sum(agg * agg, axis=1, keepdims=True))
        o_ref[...] = agg / jnp.maximum(norm, 1e-12)

    return pl.pallas_call(
        fin,
        grid=(4,),
        in_specs=[pl.BlockSpec((E_ROWS // 4, DIMS), lambda i: (i, 0)),
                  pl.BlockSpec((E_ROWS // 4, 2), lambda i: (i, 0))],
        out_specs=pl.BlockSpec((E_ROWS // 4, DIMS), lambda i: (i, 0)),
        out_shape=jax.ShapeDtypeStruct((E_ROWS, DIMS), jnp.float32),
    )(acc, den2)


def kernel(layers_num, user_emb, entity_emb, inter_edge, inter_edge_w,
           edge_index, edge_type, relation_emb, W_Q):
    f32 = jnp.float32
    headp = jnp.concatenate([
        edge_index[0].astype(jnp.int32),
        jnp.full((NEP - N_EDGES,), N_ENTITIES, jnp.int32)])
    tailp = jnp.concatenate([
        edge_index[1].astype(jnp.int32),
        jnp.zeros((NEP - N_EDGES,), jnp.int32)])
    etp = jnp.concatenate([
        edge_type.astype(jnp.int32),
        jnp.zeros((NEP - N_EDGES,), jnp.int32)])
    iu = jnp.concatenate([
        inter_edge[0].astype(jnp.int32),
        jnp.full((NUP - N_INTER,), N_USERS, jnp.int32)])
    ii = jnp.concatenate([
        inter_edge[1].astype(jnp.int32),
        jnp.zeros((NUP - N_INTER,), jnp.int32)])
    iw = jnp.concatenate([
        inter_edge_w.astype(f32), jnp.zeros((NUP - N_INTER,), f32)])
    r2 = jnp.roll(relation_emb.astype(f32), 1, axis=0)
    zrows = jnp.zeros((E_ROWS, DIMS), f32)

    e_tab = jnp.concatenate(
        [entity_emb.astype(f32),
         jnp.zeros((E_ROWS - N_ENTITIES, DIMS), f32)], axis=0)
    w_q = W_Q.astype(f32)

    e_hist = [e_tab]
    u_hist = [user_emb.astype(f32)]
    for _ in range(LAYERS):
        p_tab = _tc_project(e_hist[-1], w_q)
        acc, den, uacc = _sc_agg(p_tab, e_hist[-1], r2, headp, tailp, etp,
                                 iu, ii, iw, zrows)
        den2 = den.reshape(DEN_ROWS * D_K, N_HEADS)[:E_ROWS]
        e_hist.append(_tc_finalize(acc, den2))
        u_hist.append(uacc[:N_USERS, :DIMS])

    inv = f32(1.0 / (LAYERS + 1))
    entity_out = (e_hist[0][:N_ENTITIES] + e_hist[1][:N_ENTITIES]
                  + e_hist[2][:N_ENTITIES]) * inv
    user_out = (u_hist[0] + u_hist[1] + u_hist[2]) * inv
    del layers_num
    return (user_out, entity_out)
